# Initial kernel scaffold; baseline (speedup 1.0000x reference)
import jax
import jax.numpy as jnp
from jax.experimental import pallas as pl


def kernel(scores):
    raise NotImplementedError("write your pallas kernel here")


# speedup vs baseline: 3.4996x; 3.4996x over previous
import functools
import math

import jax
import jax.numpy as jnp
from jax import lax
from jax.experimental import pallas as pl
from jax.experimental.pallas import tpu as pltpu
from jax.experimental.pallas import tpu_sc as plsc

_K = 8
_N = 64
_LANES = 16
_NUM_CORES = 2
_NUM_SUBCORES = 16
_NUM_WORKERS = _NUM_CORES * _NUM_SUBCORES


def _sc_body(theta_hbm, u_hbm, mask_hbm, marg_hbm, th_v, u_v, w_v, btab,
             mask_v, marg_v, *, groups_per_worker):
    wid = lax.axis_index("s") * _NUM_CORES + lax.axis_index("c")
    lane = jnp.arange(_LANES, dtype=jnp.int32)
    one = jnp.ones((_LANES,), jnp.float32)
    zero = jnp.zeros((_LANES,), jnp.float32)

    def do_group(i, carry):
        g = wid * groups_per_worker + i
        pltpu.sync_copy(theta_hbm.at[g], th_v)
        pltpu.sync_copy(u_hbm.at[g], u_v)

        acc = th_v[0]
        for j in range(1, _N):
            acc = acc + th_v[j]
        mu = acc * jnp.float32(1.0 / _N)
        for j in range(_N):
            w_v[j] = jnp.exp(th_v[j] - mu)

        bs = [one] + [zero] * _K
        for r in range(_K + 1):
            btab[_N * (_K + 1) + r] = bs[r]
        for j in range(_N - 1, -1, -1):
            wj = w_v[j]
            for r in range(_K, 0, -1):
                bs[r] = bs[r] + wj * bs[r - 1]
            for r in range(_K + 1):
                btab[j * (_K + 1) + r] = bs[r]

        inv_ek = one / btab[_K]

        fs = [one] + [zero] * (_K - 1)
        rem = jnp.full((_LANES,), _K, jnp.int32)
        for j in range(_N):
            wj = w_v[j]
            base1 = (j + 1) * (_K + 1)
            dot = fs[0] * btab[base1 + _K - 1]
            for r in range(1, _K):
                dot = dot + fs[r] * btab[base1 + _K - 1 - r]
            marg_v[j] = wj * dot * inv_ek
            for r in range(_K - 1, 0, -1):
                fs[r] = fs[r] + wj * fs[r - 1]

            jv = jnp.full((_LANES,), j * (_K + 1), jnp.int32)
            jv1 = jnp.full((_LANES,), base1, jnp.int32)
            b_cur = plsc.load_gather(btab, [jv + rem, lane])
            b_inc = plsc.load_gather(btab, [jv1 + jnp.maximum(rem - 1, 0), lane])
            p = wj * b_inc / jnp.maximum(b_cur, jnp.float32(1e-35))
            p = jnp.minimum(p, jnp.float32(1.0))
            p = jnp.where(rem > 0, p, jnp.float32(0.0))
            inc = u_v[j] < p
            rem = rem - inc.astype(jnp.int32)
            mask_v[j] = inc.astype(jnp.float32)

        pltpu.sync_copy(mask_v, mask_hbm.at[g])
        pltpu.sync_copy(marg_v, marg_hbm.at[g])
        return carry

    lax.fori_loop(0, groups_per_worker, do_group, 0)


@jax.jit
def _sc_sampler(theta3, u3):
    g_total = theta3.shape[0]
    groups_per_worker = g_total // _NUM_WORKERS
    mesh = plsc.VectorSubcoreMesh(
        core_axis_name="c", subcore_axis_name="s",
        num_cores=_NUM_CORES, num_subcores=_NUM_SUBCORES)
    body = functools.partial(_sc_body, groups_per_worker=groups_per_worker)
    f = pl.kernel(
        body,
        out_type=(
            jax.ShapeDtypeStruct((g_total, _N, _LANES), jnp.float32),
            jax.ShapeDtypeStruct((g_total, _N, _LANES), jnp.float32),
        ),
        mesh=mesh,
        compiler_params=pltpu.CompilerParams(needs_layout_passes=False),
        scratch_types=[
            pltpu.VMEM((_N, _LANES), jnp.float32),
            pltpu.VMEM((_N, _LANES), jnp.float32),
            pltpu.VMEM((_N, _LANES), jnp.float32),
            pltpu.VMEM(((_N + 1) * (_K + 1), _LANES), jnp.float32),
            pltpu.VMEM((_N, _LANES), jnp.float32),
            pltpu.VMEM((_N, _LANES), jnp.float32),
        ],
    )
    return f(theta3, u3)


def kernel(scores):
    nnodes, choices, ensemble = scores.shape
    rows = nnodes * ensemble
    groups = rows // _LANES
    flat = jnp.transpose(scores, (0, 2, 1)).reshape(rows, choices)
    theta3 = flat.reshape(groups, _LANES, choices).transpose(0, 2, 1)
    uni = jax.random.uniform(jax.random.key(42), (choices, 1, rows),
                             jnp.float32)
    u3 = uni.reshape(choices, groups, _LANES).transpose(1, 0, 2)

    mask3, marg3 = _sc_sampler(theta3, u3)

    samples = mask3.transpose(0, 2, 1).reshape(rows, choices)
    marg = marg3.transpose(0, 2, 1).reshape(rows, choices)
    new_mask = samples.reshape(1, nnodes, ensemble, choices)
    new_mask = jnp.transpose(new_mask, (0, 1, 3, 2))
    new_marg = jnp.transpose(marg.reshape(nnodes, ensemble, choices),
                             (0, 2, 1))
    return new_mask, new_marg
